# baseline (speedup 1.0000x reference)
import functools

import jax
import jax.numpy as jnp
from jax import lax
from jax.experimental import pallas as pl
from jax.experimental.pallas import tpu as pltpu
from jax.experimental.pallas import tpu_sc as plsc

EMBED_DIM = 64
BATCH = 16384
NUM_CORES = 2
NUM_SUBCORES = 16
NUM_WORKERS = NUM_CORES * NUM_SUBCORES
BPW = BATCH // NUM_WORKERS
CHUNK = 128
NCHUNK = BPW // CHUNK
NGROUPS = BPW // 16


def _sc_body(heads2, rels2, tails2, ent_hbm, rel_hbm, wc_hbm, wr_hbm, b_hbm,
             out_hbm,
             hidx, ridx, tidx, hrows, rrows, trows, wc_v, wr_v, b_v, tot_v,
             out_v, sem):
    c = lax.axis_index("c")
    s = lax.axis_index("s")
    wid = s * NUM_CORES + c

    pltpu.sync_copy(heads2.at[pl.ds(wid * NCHUNK, NCHUNK)], hidx)
    pltpu.sync_copy(rels2.at[pl.ds(wid * NCHUNK, NCHUNK)], ridx)
    pltpu.sync_copy(tails2.at[pl.ds(wid * NCHUNK, NCHUNK)], tidx)
    pltpu.sync_copy(wc_hbm, wc_v)
    pltpu.sync_copy(wr_hbm, wr_v)
    pltpu.sync_copy(b_hbm, b_v)

    copies = []
    for j in range(NCHUNK):
        dst = pl.ds(j * CHUNK, CHUNK)
        copies.append(pltpu.async_copy(ent_hbm.at[hidx.at[j]], hrows.at[dst], sem))
        copies.append(pltpu.async_copy(ent_hbm.at[tidx.at[j]], trows.at[dst], sem))
        copies.append(pltpu.async_copy(rel_hbm.at[ridx.at[j]], rrows.at[dst], sem))
    for cp in copies:
        cp.wait()

    iota16 = lax.iota(jnp.int32, 16)
    bvec = b_v[...]
    wcv = [wc_v[pl.ds(k * 16, 16)] for k in range(4)]
    wrv = [wr_v[pl.ds(k * 16, 16)] for k in range(4)]
    lane15 = iota16 * 16 + 15

    def group(g, carry):
        row0 = g * 16
        for j in range(16):
            row = row0 + j
            acc = jnp.zeros((16,), jnp.float32)
            for k in range(4):
                blk = pl.ds(k * 16, 16)
                hv = hrows[row, blk]
                tv = trows[row, blk]
                rv = rrows[row, blk]
                acc = acc + (hv - tv) * wcv[k] + rv * wrv[k]
            tot_v[pl.ds(j * 16, 16)] = plsc.cumsum(acc)
        out_v[pl.ds(row0, 16)] = plsc.load_gather(tot_v, [lane15]) + bvec
        return carry

    lax.fori_loop(0, NGROUPS, group, 0)

    pltpu.sync_copy(out_v, out_hbm.at[pl.ds(wid * BPW, BPW)])


_sc_call = functools.partial(
    pl.kernel,
    out_type=jax.ShapeDtypeStruct((BATCH,), jnp.float32),
    mesh=plsc.VectorSubcoreMesh(core_axis_name="c", subcore_axis_name="s",
                                num_cores=NUM_CORES,
                                num_subcores=NUM_SUBCORES),
    compiler_params=pltpu.CompilerParams(needs_layout_passes=False,
                                         use_tc_tiling_on_sc=False),
    scratch_types=[
        pltpu.VMEM((NCHUNK, CHUNK), jnp.int32),
        pltpu.VMEM((NCHUNK, CHUNK), jnp.int32),
        pltpu.VMEM((NCHUNK, CHUNK), jnp.int32),
        pltpu.VMEM((BPW, EMBED_DIM), jnp.float32),
        pltpu.VMEM((BPW, EMBED_DIM), jnp.float32),
        pltpu.VMEM((BPW, EMBED_DIM), jnp.float32),
        pltpu.VMEM((EMBED_DIM,), jnp.float32),
        pltpu.VMEM((EMBED_DIM,), jnp.float32),
        pltpu.VMEM((16,), jnp.float32),
        pltpu.VMEM((256,), jnp.float32),
        pltpu.VMEM((BPW,), jnp.float32),
        pltpu.SemaphoreType.DMA,
    ],
)(_sc_body)


def kernel(heads, relations, tails, entity_emb, relation_emb, W, b):
    heads2 = heads.astype(jnp.int32).reshape(NUM_WORKERS * NCHUNK, CHUNK)
    rels2 = relations.astype(jnp.int32).reshape(NUM_WORKERS * NCHUNK, CHUNK)
    tails2 = tails.astype(jnp.int32).reshape(NUM_WORKERS * NCHUNK, CHUNK)
    w = W.reshape(2, EMBED_DIM).astype(jnp.float32)
    wc = w[0] + w[1]
    wr = w[0]
    bb = jnp.broadcast_to(b.astype(jnp.float32), (16,))
    return _sc_call(heads2, rels2, tails2, entity_emb, relation_emb,
                    wc, wr, bb)


# speedup vs baseline: 1.0006x; 1.0006x over previous
import functools

import jax
import jax.numpy as jnp
from jax import lax
from jax.experimental import pallas as pl
from jax.experimental.pallas import tpu as pltpu
from jax.experimental.pallas import tpu_sc as plsc

EMBED_DIM = 64
BATCH = 16384
NUM_CORES = 2
NUM_SUBCORES = 16
NUM_WORKERS = NUM_CORES * NUM_SUBCORES
BPW = BATCH // NUM_WORKERS
CHUNK = 128
NCHUNK = BPW // CHUNK
GPC = CHUNK // 16


def _sc_body(heads2, rels2, tails2, ent_hbm, rel_hbm, wc_hbm, wr_hbm, b_hbm,
             out_hbm,
             hidx, ridx, tidx, hrows, rrows, trows, wc_v, wr_v, b_v, tot_v,
             out_v, sems):
    c = lax.axis_index("c")
    s = lax.axis_index("s")
    wid = s * NUM_CORES + c

    pltpu.sync_copy(heads2.at[pl.ds(wid * NCHUNK, NCHUNK)], hidx)
    pltpu.sync_copy(rels2.at[pl.ds(wid * NCHUNK, NCHUNK)], ridx)
    pltpu.sync_copy(tails2.at[pl.ds(wid * NCHUNK, NCHUNK)], tidx)
    pltpu.sync_copy(wc_hbm, wc_v)
    pltpu.sync_copy(wr_hbm, wr_v)
    pltpu.sync_copy(b_hbm, b_v)

    copies = []
    for j in range(NCHUNK):
        dst = pl.ds(j * CHUNK, CHUNK)
        copies.append((
            pltpu.async_copy(ent_hbm.at[hidx.at[j]], hrows.at[dst], sems[j]),
            pltpu.async_copy(ent_hbm.at[tidx.at[j]], trows.at[dst], sems[j]),
            pltpu.async_copy(rel_hbm.at[ridx.at[j]], rrows.at[dst], sems[j]),
        ))

    iota16 = lax.iota(jnp.int32, 16)
    bvec = b_v[...]
    wcv = [wc_v[pl.ds(k * 16, 16)] for k in range(4)]
    wrv = [wr_v[pl.ds(k * 16, 16)] for k in range(4)]
    stride16 = iota16 * 16

    def group(g, carry):
        row0 = g * 16
        for j in range(16):
            row = row0 + j
            acc = jnp.zeros((16,), jnp.float32)
            for k in range(4):
                blk = pl.ds(k * 16, 16)
                hv = hrows[row, blk]
                tv = trows[row, blk]
                rv = rrows[row, blk]
                acc = acc + (hv - tv) * wcv[k] + rv * wrv[k]
            tot_v[pl.ds(j * 16, 16)] = acc
        res = bvec
        for l in range(16):
            res = res + plsc.load_gather(tot_v, [stride16 + l])
        out_v[pl.ds(row0, 16)] = res
        return carry

    for j in range(NCHUNK):
        for cp in copies[j]:
            cp.wait()
        lax.fori_loop(j * GPC, (j + 1) * GPC, group, 0)

    pltpu.sync_copy(out_v, out_hbm.at[pl.ds(wid * BPW, BPW)])


_sc_call = functools.partial(
    pl.kernel,
    out_type=jax.ShapeDtypeStruct((BATCH,), jnp.float32),
    mesh=plsc.VectorSubcoreMesh(core_axis_name="c", subcore_axis_name="s",
                                num_cores=NUM_CORES,
                                num_subcores=NUM_SUBCORES),
    compiler_params=pltpu.CompilerParams(needs_layout_passes=False,
                                         use_tc_tiling_on_sc=False),
    scratch_types=[
        pltpu.VMEM((NCHUNK, CHUNK), jnp.int32),
        pltpu.VMEM((NCHUNK, CHUNK), jnp.int32),
        pltpu.VMEM((NCHUNK, CHUNK), jnp.int32),
        pltpu.VMEM((BPW, EMBED_DIM), jnp.float32),
        pltpu.VMEM((BPW, EMBED_DIM), jnp.float32),
        pltpu.VMEM((BPW, EMBED_DIM), jnp.float32),
        pltpu.VMEM((EMBED_DIM,), jnp.float32),
        pltpu.VMEM((EMBED_DIM,), jnp.float32),
        pltpu.VMEM((16,), jnp.float32),
        pltpu.VMEM((256,), jnp.float32),
        pltpu.VMEM((BPW,), jnp.float32),
        [pltpu.SemaphoreType.DMA] * NCHUNK,
    ],
)(_sc_body)


def kernel(heads, relations, tails, entity_emb, relation_emb, W, b):
    heads2 = heads.astype(jnp.int32).reshape(NUM_WORKERS * NCHUNK, CHUNK)
    rels2 = relations.astype(jnp.int32).reshape(NUM_WORKERS * NCHUNK, CHUNK)
    tails2 = tails.astype(jnp.int32).reshape(NUM_WORKERS * NCHUNK, CHUNK)
    w = W.reshape(2, EMBED_DIM).astype(jnp.float32)
    wc = w[0] + w[1]
    wr = w[0]
    bb = jnp.broadcast_to(b.astype(jnp.float32), (16,))
    return _sc_call(heads2, rels2, tails2, entity_emb, relation_emb,
                    wc, wr, bb)
